# baseline (speedup 1.0000x reference)
import jax
import jax.numpy as jnp
from jax.experimental import pallas as pl
from jax.experimental.pallas import tpu as pltpu

_NUM_EXPERTS = 64
_TOP_K = 8
_MODEL_DIM = 2048
_T = 16384
_BLOCK = 512


def _router_kernel(x_ref, w_ref, probs_ref, weights_ref, idx_ref):
    x = x_ref[...]
    w = w_ref[...]
    logits = jax.lax.dot_general(
        x, w, (((1,), (1,)), ((), ())), preferred_element_type=jnp.float32
    )
    m = jnp.max(logits, axis=-1, keepdims=True)
    e = jnp.exp(logits - m)
    s = jnp.sum(e, axis=-1, keepdims=True)
    probs = e / s
    probs_ref[...] = probs

    B = probs.shape[0]
    lane = jax.lax.broadcasted_iota(jnp.int32, (B, _NUM_EXPERTS), 1)
    col = jax.lax.broadcasted_iota(jnp.int32, (B, _TOP_K), 1)
    pm = probs
    vals = jnp.zeros((B, _TOP_K), jnp.float32)
    idxs = jnp.zeros((B, _TOP_K), jnp.int32)
    for j in range(_TOP_K):
        mj = jnp.max(pm, axis=-1, keepdims=True)
        eq = pm == mj
        ij = jnp.min(jnp.where(eq, lane, _NUM_EXPERTS), axis=-1,
                     keepdims=True)
        vals = jnp.where(col == j, mj, vals)
        idxs = jnp.where(col == j, ij, idxs)
        pm = jnp.where(lane == ij, -jnp.inf, pm)
    weights_ref[...] = vals / jnp.sum(vals, axis=-1, keepdims=True)
    idx_ref[...] = idxs


def kernel(hidden_states, weight):
    x = hidden_states.reshape(-1, _MODEL_DIM)
    T = x.shape[0]
    grid = (T // _BLOCK,)
    probs, weights, idxs = pl.pallas_call(
        _router_kernel,
        grid=grid,
        in_specs=[
            pl.BlockSpec((_BLOCK, _MODEL_DIM), lambda i: (i, 0)),
            pl.BlockSpec((_NUM_EXPERTS, _MODEL_DIM), lambda i: (0, 0)),
        ],
        out_specs=[
            pl.BlockSpec((_BLOCK, _NUM_EXPERTS), lambda i: (i, 0)),
            pl.BlockSpec((_BLOCK, _TOP_K), lambda i: (i, 0)),
            pl.BlockSpec((_BLOCK, _TOP_K), lambda i: (i, 0)),
        ],
        out_shape=[
            jax.ShapeDtypeStruct((T, _NUM_EXPERTS), jnp.float32),
            jax.ShapeDtypeStruct((T, _TOP_K), jnp.float32),
            jax.ShapeDtypeStruct((T, _TOP_K), jnp.int32),
        ],
        compiler_params=pltpu.CompilerParams(
            dimension_semantics=("arbitrary",),
        ),
    )(x, weight)
    return (probs, weights, idxs)


# speedup vs baseline: 1.4162x; 1.4162x over previous
import jax
import jax.numpy as jnp
from jax.experimental import pallas as pl
from jax.experimental.pallas import tpu as pltpu

_NUM_EXPERTS = 64
_TOP_K = 8
_MODEL_DIM = 2048
_BLOCK = 512


def _router_kernel(x_ref, w_ref, probs_ref, weights_ref, idx_ref):
    x = x_ref[...]
    w = w_ref[...]
    logits = jax.lax.dot_general(
        w, x, (((1,), (1,)), ((), ())), preferred_element_type=jnp.float32
    )
    m = jnp.max(logits, axis=0, keepdims=True)
    e = jnp.exp(logits - m)
    s = jnp.sum(e, axis=0, keepdims=True)
    probs = e / s
    probs_ref[...] = probs.T

    B = probs.shape[1]
    expert = jax.lax.broadcasted_iota(jnp.int32, (_NUM_EXPERTS, B), 0)
    pm = probs
    vals = []
    idxs = []
    for _ in range(_TOP_K):
        mj = jnp.max(pm, axis=0, keepdims=True)
        eq = pm == mj
        ij = jnp.min(jnp.where(eq, expert, _NUM_EXPERTS), axis=0,
                     keepdims=True)
        vals.append(mj)
        idxs.append(ij)
        pm = jnp.where(expert == ij, -jnp.inf, pm)
    v = jnp.concatenate(vals, axis=0)
    i = jnp.concatenate(idxs, axis=0)
    v = v / jnp.sum(v, axis=0, keepdims=True)
    weights_ref[...] = v.T
    idx_ref[...] = i.T


def kernel(hidden_states, weight):
    x = hidden_states.reshape(-1, _MODEL_DIM)
    T = x.shape[0]
    grid = (T // _BLOCK,)
    probs, weights, idxs = pl.pallas_call(
        _router_kernel,
        grid=grid,
        in_specs=[
            pl.BlockSpec((_BLOCK, _MODEL_DIM), lambda i: (i, 0)),
            pl.BlockSpec((_NUM_EXPERTS, _MODEL_DIM), lambda i: (0, 0)),
        ],
        out_specs=[
            pl.BlockSpec((_BLOCK, _NUM_EXPERTS), lambda i: (i, 0)),
            pl.BlockSpec((_BLOCK, _TOP_K), lambda i: (i, 0)),
            pl.BlockSpec((_BLOCK, _TOP_K), lambda i: (i, 0)),
        ],
        out_shape=[
            jax.ShapeDtypeStruct((T, _NUM_EXPERTS), jnp.float32),
            jax.ShapeDtypeStruct((T, _TOP_K), jnp.float32),
            jax.ShapeDtypeStruct((T, _TOP_K), jnp.int32),
        ],
        compiler_params=pltpu.CompilerParams(
            dimension_semantics=("arbitrary",),
        ),
    )(x, weight)
    return (probs, weights, idxs)


# speedup vs baseline: 1.6324x; 1.1527x over previous
import jax
import jax.numpy as jnp
from jax.experimental import pallas as pl
from jax.experimental.pallas import tpu as pltpu

_NUM_EXPERTS = 64
_TOP_K = 8
_MODEL_DIM = 2048
_BLOCK = 1024


def _router_kernel(x_ref, w_ref, probs_ref, weights_ref, idx_ref):
    x = x_ref[...]
    w = w_ref[...]
    logits = jax.lax.dot_general(
        w, x, (((1,), (1,)), ((), ())), preferred_element_type=jnp.float32
    )
    m = jnp.max(logits, axis=0, keepdims=True)
    e = jnp.exp(logits - m)
    s = jnp.sum(e, axis=0, keepdims=True)
    probs = e / s
    probs_ref[...] = probs.T

    B = probs.shape[1]
    expert = jax.lax.broadcasted_iota(jnp.int32, (_NUM_EXPERTS, B), 0)
    pm = probs
    vals = []
    idxs = []
    for _ in range(_TOP_K):
        mj = jnp.max(pm, axis=0, keepdims=True)
        eq = pm == mj
        ij = jnp.min(jnp.where(eq, expert, _NUM_EXPERTS), axis=0,
                     keepdims=True)
        vals.append(mj)
        idxs.append(ij)
        pm = jnp.where(expert == ij, -jnp.inf, pm)
    v = jnp.concatenate(vals, axis=0)
    i = jnp.concatenate(idxs, axis=0)
    v = v / jnp.sum(v, axis=0, keepdims=True)
    weights_ref[...] = v.T
    idx_ref[...] = i.T


def kernel(hidden_states, weight):
    x = hidden_states.reshape(-1, _MODEL_DIM)
    T = x.shape[0]
    grid = (T // _BLOCK,)
    probs, weights, idxs = pl.pallas_call(
        _router_kernel,
        grid=grid,
        in_specs=[
            pl.BlockSpec((_BLOCK, _MODEL_DIM), lambda i: (i, 0)),
            pl.BlockSpec((_NUM_EXPERTS, _MODEL_DIM), lambda i: (0, 0)),
        ],
        out_specs=[
            pl.BlockSpec((_BLOCK, _NUM_EXPERTS), lambda i: (i, 0)),
            pl.BlockSpec((_BLOCK, _TOP_K), lambda i: (i, 0)),
            pl.BlockSpec((_BLOCK, _TOP_K), lambda i: (i, 0)),
        ],
        out_shape=[
            jax.ShapeDtypeStruct((T, _NUM_EXPERTS), jnp.float32),
            jax.ShapeDtypeStruct((T, _TOP_K), jnp.float32),
            jax.ShapeDtypeStruct((T, _TOP_K), jnp.int32),
        ],
        compiler_params=pltpu.CompilerParams(
            dimension_semantics=("arbitrary",),
        ),
    )(x, weight)
    return (probs, weights, idxs)


# speedup vs baseline: 1.6960x; 1.0390x over previous
import jax
import jax.numpy as jnp
from jax.experimental import pallas as pl
from jax.experimental.pallas import tpu as pltpu

_NUM_EXPERTS = 64
_TOP_K = 8
_MODEL_DIM = 2048
_BLOCK = 2048


def _router_kernel(x_ref, w_ref, probs_ref, weights_ref, idx_ref):
    x = x_ref[...]
    w = w_ref[...]
    logits = jax.lax.dot_general(
        w, x, (((1,), (1,)), ((), ())), preferred_element_type=jnp.float32
    )
    m = jnp.max(logits, axis=0, keepdims=True)
    e = jnp.exp(logits - m)
    s = jnp.sum(e, axis=0, keepdims=True)
    probs = e / s
    probs_ref[...] = probs.T

    B = probs.shape[1]
    expert = jax.lax.broadcasted_iota(jnp.int32, (_NUM_EXPERTS, B), 0)
    pm = probs
    vals = []
    idxs = []
    for _ in range(_TOP_K):
        mj = jnp.max(pm, axis=0, keepdims=True)
        eq = pm == mj
        ij = jnp.min(jnp.where(eq, expert, _NUM_EXPERTS), axis=0,
                     keepdims=True)
        vals.append(mj)
        idxs.append(ij)
        pm = jnp.where(expert == ij, -jnp.inf, pm)
    v = jnp.concatenate(vals, axis=0)
    i = jnp.concatenate(idxs, axis=0)
    v = v / jnp.sum(v, axis=0, keepdims=True)
    weights_ref[...] = v.T
    idx_ref[...] = i.T


def kernel(hidden_states, weight):
    x = hidden_states.reshape(-1, _MODEL_DIM)
    T = x.shape[0]
    grid = (T // _BLOCK,)
    probs, weights, idxs = pl.pallas_call(
        _router_kernel,
        grid=grid,
        in_specs=[
            pl.BlockSpec((_BLOCK, _MODEL_DIM), lambda i: (i, 0)),
            pl.BlockSpec((_NUM_EXPERTS, _MODEL_DIM), lambda i: (0, 0)),
        ],
        out_specs=[
            pl.BlockSpec((_BLOCK, _NUM_EXPERTS), lambda i: (i, 0)),
            pl.BlockSpec((_BLOCK, _TOP_K), lambda i: (i, 0)),
            pl.BlockSpec((_BLOCK, _TOP_K), lambda i: (i, 0)),
        ],
        out_shape=[
            jax.ShapeDtypeStruct((T, _NUM_EXPERTS), jnp.float32),
            jax.ShapeDtypeStruct((T, _TOP_K), jnp.float32),
            jax.ShapeDtypeStruct((T, _TOP_K), jnp.int32),
        ],
        compiler_params=pltpu.CompilerParams(
            dimension_semantics=("arbitrary",),
        ),
    )(x, weight)
    return (probs, weights, idxs)
